# baseline (speedup 1.0000x reference)
import functools

import jax
import jax.numpy as jnp
import numpy as np
from jax import lax
from jax.experimental import pallas as pl
from jax.experimental.pallas import tpu as pltpu
from jax.experimental.pallas import tpu_sc as plsc

_B = 1024
_L = 200
_D = 128
_SCALE = float(np.sqrt(np.float32(_D)))

_NC = 2
_NS = 16
_NW = _NC * _NS
_SEQ_PER_W = _B // _NW


def _positional_table():
    pos = np.arange(_L)[:, None].astype(np.float32)
    i = np.arange(_D)[None, :].astype(np.float32)
    angle_rates = 1.0 / np.power(
        10000.0, (2.0 * np.floor(i / 2.0)) / np.float32(_D))
    angles = pos * angle_rates
    pe = np.zeros((_L, _D), dtype=np.float32)
    pe[:, 0::2] = np.sin(angles[:, 0::2])
    pe[:, 1::2] = np.cos(angles[:, 1::2])
    return pe


_PE = _positional_table()


def _enc_kernel(idx_hbm, table_hbm, pe_hbm, out_hbm,
                pe_v, idx_a, idx_b, rows_v, sem):
    wid = lax.axis_index("s") * _NC + lax.axis_index("c")
    pltpu.sync_copy(pe_hbm, pe_v)

    def one_seq(j, carry):
        base = (wid * _SEQ_PER_W + j) * _L
        pltpu.sync_copy(idx_hbm.at[pl.ds(base, 128)], idx_a)
        pltpu.sync_copy(idx_hbm.at[pl.ds(base + 128, _L - 128)], idx_b)
        cp_a = pltpu.async_copy(table_hbm.at[idx_a],
                                rows_v.at[pl.ds(0, 128)], sem)
        cp_b = pltpu.async_copy(table_hbm.at[idx_b],
                                rows_v.at[pl.ds(128, _L - 128)], sem)
        cp_a.wait()
        cp_b.wait()

        def fuse(l, c):
            for d in range(_D // 16):
                sl = pl.ds(d * 16, 16)
                rows_v[l, sl] = rows_v[l, sl] * _SCALE + pe_v[l, sl]
            return c

        lax.fori_loop(0, _L, fuse, 0)
        pltpu.sync_copy(rows_v, out_hbm.at[pl.ds(base, _L)])
        return carry

    lax.fori_loop(0, _SEQ_PER_W, one_seq, 0)


@jax.jit
def _encode(idx, emb_table, pe):
    mesh = plsc.VectorSubcoreMesh(core_axis_name="c", subcore_axis_name="s")
    f = functools.partial(
        pl.kernel,
        out_type=jax.ShapeDtypeStruct((_B * _L, _D), jnp.float32),
        mesh=mesh,
        scratch_types=[
            pltpu.VMEM((_L, _D), jnp.float32),
            pltpu.VMEM((128,), jnp.int32),
            pltpu.VMEM((_L - 128,), jnp.int32),
            pltpu.VMEM((_L, _D), jnp.float32),
            pltpu.SemaphoreType.DMA,
        ],
    )(_enc_kernel)
    return f(idx, emb_table, pe)


def kernel(text, emb_table):
    idx = text.reshape(-1).astype(jnp.int32)
    out = _encode(idx, emb_table, _PE)
    return out.reshape(_B, _L, _D)


# speedup vs baseline: 1.9652x; 1.9652x over previous
import functools

import jax
import jax.numpy as jnp
import numpy as np
from jax import lax
from jax.experimental import pallas as pl
from jax.experimental.pallas import tpu as pltpu
from jax.experimental.pallas import tpu_sc as plsc

_B = 1024
_L = 200
_D = 128
_SCALE = float(np.sqrt(np.float32(_D)))

_NC = 2
_NS = 16
_NW = _NC * _NS
_SEQ_PER_W = _B // _NW
_NBUF = 3


def _positional_table():
    pos = np.arange(_L)[:, None].astype(np.float32)
    i = np.arange(_D)[None, :].astype(np.float32)
    angle_rates = 1.0 / np.power(
        10000.0, (2.0 * np.floor(i / 2.0)) / np.float32(_D))
    angles = pos * angle_rates
    pe = np.zeros((_L, _D), dtype=np.float32)
    pe[:, 0::2] = np.sin(angles[:, 0::2])
    pe[:, 1::2] = np.cos(angles[:, 1::2])
    return pe


_PE = _positional_table()


def _enc_kernel(idx_hbm, table_hbm, pe_hbm, out_hbm,
                pe_v, idx_v, r0, r1, r2, gsem, wsem):
    wid = lax.axis_index("s") * _NC + lax.axis_index("c")
    rows = (r0, r1, r2)
    pltpu.sync_copy(pe_hbm, pe_v)
    pltpu.sync_copy(idx_hbm.at[pl.ds(wid * _SEQ_PER_W * _L, _SEQ_PER_W * _L)],
                    idx_v)

    def issue_gather(j, b):
        off = j * _L
        pltpu.async_copy(table_hbm.at[idx_v.at[pl.ds(off, 128)]],
                         rows[b].at[pl.ds(0, 128)], gsem.at[b])
        pltpu.async_copy(table_hbm.at[idx_v.at[pl.ds(off + 128, _L - 128)]],
                         rows[b].at[pl.ds(128, _L - 128)], gsem.at[b])

    def wait_gather(b):
        pltpu.make_async_copy(out_hbm.at[pl.ds(0, _L)], rows[b],
                              gsem.at[b]).wait()

    def issue_write(j, b):
        pltpu.async_copy(rows[b], out_hbm.at[pl.ds((wid * _SEQ_PER_W + j) * _L,
                                                   _L)], wsem.at[b])

    def wait_write(b):
        pltpu.make_async_copy(rows[b], out_hbm.at[pl.ds(0, _L)],
                              wsem.at[b]).wait()

    def fuse(b):
        def body(l, c):
            for d in range(_D // 16):
                sl = pl.ds(d * 16, 16)
                rows[b][l, sl] = rows[b][l, sl] * _SCALE + pe_v[l, sl]
            return c
        lax.fori_loop(0, _L, body, 0)

    def step(j, b, first, last):
        if not first:
            wait_write((b + 1) % _NBUF)
        if not last:
            issue_gather(j + 1, (b + 1) % _NBUF)
        wait_gather(b)
        fuse(b)
        issue_write(j, b)

    issue_gather(0, 0)

    def group(g, c):
        j0 = g * _NBUF
        for u in range(_NBUF):
            step(j0 + u, u, False, False)
        return c

    for u in range(_NBUF):
        step(u, u, u < _NBUF - 1, False)
    lax.fori_loop(1, _SEQ_PER_W // _NBUF, group, 0)
    step(_SEQ_PER_W - 2, (_SEQ_PER_W - 2) % _NBUF, False, False)
    step(_SEQ_PER_W - 1, (_SEQ_PER_W - 1) % _NBUF, False, True)
    wait_write((_SEQ_PER_W - 2) % _NBUF)
    wait_write((_SEQ_PER_W - 1) % _NBUF)


@jax.jit
def _encode(idx, emb_table, pe):
    mesh = plsc.VectorSubcoreMesh(core_axis_name="c", subcore_axis_name="s")
    f = functools.partial(
        pl.kernel,
        out_type=jax.ShapeDtypeStruct((_B * _L, _D), jnp.float32),
        mesh=mesh,
        scratch_types=[
            pltpu.VMEM((_L, _D), jnp.float32),
            pltpu.VMEM((_SEQ_PER_W * _L,), jnp.int32),
            pltpu.VMEM((_L, _D), jnp.float32),
            pltpu.VMEM((_L, _D), jnp.float32),
            pltpu.VMEM((_L, _D), jnp.float32),
            pltpu.SemaphoreType.DMA((_NBUF,)),
            pltpu.SemaphoreType.DMA((_NBUF,)),
        ],
    )(_enc_kernel)
    return f(idx, emb_table, pe)


def kernel(text, emb_table):
    idx = text.reshape(-1).astype(jnp.int32)
    out = _encode(idx, emb_table, _PE)
    return out.reshape(_B, _L, _D)
